# baseline (speedup 1.0000x reference)
import functools

import jax
import jax.numpy as jnp
from jax import lax
from jax.experimental import pallas as pl
from jax.experimental.pallas import tpu as pltpu
from jax.experimental.pallas import tpu_sc as plsc

NUM_ENT = 100000
NUM_REL = 200
D = 128
VOCAB = NUM_ENT + NUM_REL + 3
B = 4096
ARITY = 9
NUM_NEG = 128
TOK_PAD = 16

_LN_BLOCK = 1024


def _ln_body(x_ref, s_ref, b_ref, o_ref):
    x = x_ref[...]
    mu = jnp.mean(x, axis=-1, keepdims=True)
    var = jnp.mean((x - mu) ** 2, axis=-1, keepdims=True)
    o_ref[...] = (x - mu) * lax.rsqrt(var + 1e-5) * s_ref[...] + b_ref[...]


def _layernorm_table(embs, ln_scale, ln_bias):
    n = embs.shape[0]
    grid = (n + _LN_BLOCK - 1) // _LN_BLOCK
    return pl.pallas_call(
        _ln_body,
        grid=(grid,),
        in_specs=[
            pl.BlockSpec((_LN_BLOCK, D), lambda i: (i, 0)),
            pl.BlockSpec((1, D), lambda i: (0, 0)),
            pl.BlockSpec((1, D), lambda i: (0, 0)),
        ],
        out_specs=pl.BlockSpec((_LN_BLOCK, D), lambda i: (i, 0)),
        out_shape=jax.ShapeDtypeStruct((n, D), jnp.float32),
    )(embs, ln_scale.reshape(1, D), ln_bias.reshape(1, D))


def _make_sc_score():
    info = plsc.get_sparse_core_info()
    nc, ns = info.num_cores, info.num_subcores
    nw = nc * ns
    rpw = B // nw
    mesh = plsc.VectorSubcoreMesh(core_axis_name="c", subcore_axis_name="s")

    tile = 8
    ntiles = rpw // tile

    @functools.partial(
        pl.kernel,
        mesh=mesh,
        out_type=jax.ShapeDtypeStruct((B, NUM_NEG), jnp.float32),
        scratch_types=[
            pltpu.VMEM((rpw, NUM_NEG), jnp.int32),
            pltpu.VMEM((rpw * ARITY,), jnp.int32),
            pltpu.VMEM((2, tile * ARITY, D), jnp.float32),
            pltpu.VMEM((4, NUM_NEG, D), jnp.float32),
            pltpu.VMEM((rpw, NUM_NEG), jnp.float32),
            pltpu.SemaphoreType.DMA((2,)),
            pltpu.SemaphoreType.DMA((4,)),
        ],
    )
    def sc_score(unity, tok_idx, neg_idx, out, negidx_v, tokidx_v,
                 tokrows_v, negrows_v, score_v, semt, semn):
        wid = lax.axis_index("s") * nc + lax.axis_index("c")
        base = wid * rpw
        pltpu.sync_copy(neg_idx.at[pl.ds(base, rpw)], negidx_v)
        pltpu.sync_copy(tok_idx.at[pl.ds(base * ARITY, rpw * ARITY)],
                        tokidx_v)

        def issue_neg(b, s):
            for half in range(2):
                pltpu.make_async_copy(
                    unity.at[negidx_v.at[b, pl.ds(half * 64, 64)]],
                    negrows_v.at[s, pl.ds(half * 64, 64)],
                    semn.at[s]).start()

        def issue_tok(t, ts):
            pltpu.make_async_copy(
                unity.at[tokidx_v.at[pl.ds(t * tile * ARITY, tile * ARITY)]],
                tokrows_v.at[ts], semt.at[ts]).start()

        issue_tok(0, 0)
        for i in range(3):
            issue_neg(i, i)

        def row_body(b, _):
            s = b & 3
            r = b & (tile - 1)
            t = lax.shift_right_logical(b, 3)
            ts = t & 1

            @pl.when(r == 0)
            def _tok_turnover():
                pltpu.make_async_copy(
                    unity.at[tokidx_v.at[pl.ds(t * tile * ARITY,
                                               tile * ARITY)]],
                    tokrows_v.at[ts], semt.at[ts]).wait()

                @pl.when(t + 1 < ntiles)
                def _tok_prefetch():
                    issue_tok(t + 1, 1 - ts)

            @pl.when(b + 3 < rpw)
            def _neg_prefetch():
                issue_neg(b + 3, (b + 3) & 3)

            for half in range(2):
                pltpu.make_async_copy(
                    unity.at[negidx_v.at[b, pl.ds(half * 64, 64)]],
                    negrows_v.at[s, pl.ds(half * 64, 64)],
                    semn.at[s]).wait()

            h = []
            for k in range(D // 16):
                acc = tokrows_v[ts, r * ARITY, pl.ds(k * 16, 16)]
                for tt in range(1, ARITY):
                    acc = acc + tokrows_v[ts, r * ARITY + tt,
                                          pl.ds(k * 16, 16)]
                h.append(acc * (1.0 / ARITY))

            lane = lax.broadcasted_iota(jnp.int32, (16,), 0)
            bitrev = (0, 8, 4, 12, 2, 10, 6, 14, 1, 9, 5, 13, 3, 11, 7, 15)

            def _treesum(vals):
                while len(vals) > 1:
                    vals = [vals[2 * i] + vals[2 * i + 1]
                            for i in range(len(vals) // 2)]
                return vals[0]

            def _merge(a, bb, st):
                sel = (lane & st) == 0
                m = jnp.where(sel, a, bb)
                w = jnp.where(sel, bb, a)
                return m + jnp.take_along_axis(w, lane ^ st, axis=0)

            @plsc.parallel_loop(0, NUM_NEG // 16, step=1)
            def neg_body(g):
                cur = []
                for jj in bitrev:
                    j = g * 16 + jj
                    cur.append(_treesum([
                        h[k] * negrows_v[s, j, pl.ds(k * 16, 16)]
                        for k in range(D // 16)]))
                for st in (8, 4, 2, 1):
                    cur = [_merge(cur[2 * i], cur[2 * i + 1], st)
                           for i in range(len(cur) // 2)]
                score_v[b, pl.ds(g * 16, 16)] = cur[0]

            return _

        lax.fori_loop(0, rpw, row_body, None)
        pltpu.sync_copy(score_v, out.at[pl.ds(base, rpw)])

    return sc_score


_sc_score = _make_sc_score()


@jax.jit
def kernel(batch_tokens, neg_target_index, embs, ln_scale, ln_bias):
    unity = _layernorm_table(embs, ln_scale, ln_bias)
    tok = batch_tokens.astype(jnp.int32).reshape(B * ARITY)
    neg = neg_target_index.astype(jnp.int32)
    return _sc_score(unity, tok, neg)


# speedup vs baseline: 1.0020x; 1.0020x over previous
import functools

import jax
import jax.numpy as jnp
from jax import lax
from jax.experimental import pallas as pl
from jax.experimental.pallas import tpu as pltpu
from jax.experimental.pallas import tpu_sc as plsc

NUM_ENT = 100000
NUM_REL = 200
D = 128
VOCAB = NUM_ENT + NUM_REL + 3
B = 4096
ARITY = 9
NUM_NEG = 128

_LN_BLOCK = 1024


def _ln_body(x_ref, s_ref, b_ref, o_ref):
    x = x_ref[...]
    mu = jnp.mean(x, axis=-1, keepdims=True)
    var = jnp.mean((x - mu) ** 2, axis=-1, keepdims=True)
    o_ref[...] = (x - mu) * lax.rsqrt(var + 1e-5) * s_ref[...] + b_ref[...]


def _layernorm_table(embs, ln_scale, ln_bias):
    n = embs.shape[0]
    grid = (n + _LN_BLOCK - 1) // _LN_BLOCK
    return pl.pallas_call(
        _ln_body,
        grid=(grid,),
        in_specs=[
            pl.BlockSpec((_LN_BLOCK, D), lambda i: (i, 0)),
            pl.BlockSpec((1, D), lambda i: (0, 0)),
            pl.BlockSpec((1, D), lambda i: (0, 0)),
        ],
        out_specs=pl.BlockSpec((_LN_BLOCK, D), lambda i: (i, 0)),
        out_shape=jax.ShapeDtypeStruct((n, D), jnp.float32),
    )(embs, ln_scale.reshape(1, D), ln_bias.reshape(1, D))


def _make_sc_score():
    info = plsc.get_sparse_core_info()
    nc, ns = info.num_cores, info.num_subcores
    nw = nc * ns
    rpw = B // nw
    mesh = plsc.VectorSubcoreMesh(core_axis_name="c", subcore_axis_name="s")

    tile = 8
    ntiles = rpw // tile

    @functools.partial(
        pl.kernel,
        mesh=mesh,
        out_type=jax.ShapeDtypeStruct((B, NUM_NEG), jnp.float32),
        scratch_types=[
            pltpu.VMEM((rpw, NUM_NEG), jnp.int32),
            pltpu.VMEM((rpw * ARITY,), jnp.int32),
            pltpu.VMEM((2, tile * ARITY, D), jnp.float32),
            pltpu.VMEM((4, NUM_NEG, D), jnp.float32),
            pltpu.VMEM((rpw, NUM_NEG), jnp.float32),
            pltpu.SemaphoreType.DMA((2,)),
            pltpu.SemaphoreType.DMA((4,)),
        ],
    )
    def sc_score(unity, tok_idx, neg_idx, out, negidx_v, tokidx_v,
                 tokrows_v, negrows_v, score_v, semt, semn):
        wid = lax.axis_index("s") * nc + lax.axis_index("c")
        base = wid * rpw
        pltpu.sync_copy(neg_idx.at[pl.ds(base, rpw)], negidx_v)
        pltpu.sync_copy(tok_idx.at[pl.ds(base * ARITY, rpw * ARITY)],
                        tokidx_v)

        def issue_neg(b, s):
            pltpu.make_async_copy(unity.at[negidx_v.at[b]], negrows_v.at[s],
                                  semn.at[s]).start()

        def issue_tok(t, ts):
            pltpu.make_async_copy(
                unity.at[tokidx_v.at[pl.ds(t * tile * ARITY, tile * ARITY)]],
                tokrows_v.at[ts], semt.at[ts]).start()

        issue_tok(0, 0)
        for i in range(3):
            issue_neg(i, i)

        def row_body(b, _):
            s = b & 3
            r = b & (tile - 1)
            t = lax.shift_right_logical(b, 3)
            ts = t & 1

            @pl.when(r == 0)
            def _tok_turnover():
                pltpu.make_async_copy(
                    unity.at[tokidx_v.at[pl.ds(t * tile * ARITY,
                                               tile * ARITY)]],
                    tokrows_v.at[ts], semt.at[ts]).wait()

                @pl.when(t + 1 < ntiles)
                def _tok_prefetch():
                    issue_tok(t + 1, 1 - ts)

            @pl.when(b + 3 < rpw)
            def _neg_prefetch():
                issue_neg(b + 3, (b + 3) & 3)

            pltpu.make_async_copy(unity.at[negidx_v.at[b]], negrows_v.at[s],
                                  semn.at[s]).wait()

            h = []
            for k in range(D // 16):
                acc = tokrows_v[ts, r * ARITY, pl.ds(k * 16, 16)]
                for tt in range(1, ARITY):
                    acc = acc + tokrows_v[ts, r * ARITY + tt,
                                          pl.ds(k * 16, 16)]
                h.append(acc * (1.0 / ARITY))

            lane = lax.broadcasted_iota(jnp.int32, (16,), 0)
            bitrev = (0, 8, 4, 12, 2, 10, 6, 14, 1, 9, 5, 13, 3, 11, 7, 15)

            def _treesum(vals):
                while len(vals) > 1:
                    vals = [vals[2 * i] + vals[2 * i + 1]
                            for i in range(len(vals) // 2)]
                return vals[0]

            def _merge(a, bb, st):
                sel = (lane & st) == 0
                m = jnp.where(sel, a, bb)
                w = jnp.where(sel, bb, a)
                return m + jnp.take_along_axis(w, lane ^ st, axis=0)

            @plsc.parallel_loop(0, NUM_NEG // 16, step=1)
            def neg_body(g):
                cur = []
                for jj in bitrev:
                    j = g * 16 + jj
                    cur.append(_treesum([
                        h[k] * negrows_v[s, j, pl.ds(k * 16, 16)]
                        for k in range(D // 16)]))
                for st in (8, 4, 2, 1):
                    cur = [_merge(cur[2 * i], cur[2 * i + 1], st)
                           for i in range(len(cur) // 2)]
                score_v[b, pl.ds(g * 16, 16)] = cur[0]

            return _

        lax.fori_loop(0, rpw, row_body, None)
        pltpu.sync_copy(score_v, out.at[pl.ds(base, rpw)])

    return sc_score


_sc_score = _make_sc_score()


@jax.jit
def kernel(batch_tokens, neg_target_index, embs, ln_scale, ln_bias):
    unity = _layernorm_table(embs, ln_scale, ln_bias)
    tok = batch_tokens.astype(jnp.int32).reshape(B * ARITY)
    neg = neg_target_index.astype(jnp.int32)
    return _sc_score(unity, tok, neg)


# speedup vs baseline: 1.1275x; 1.1253x over previous
import functools

import jax
import jax.numpy as jnp
from jax import lax
from jax.experimental import pallas as pl
from jax.experimental.pallas import tpu as pltpu
from jax.experimental.pallas import tpu_sc as plsc

NUM_ENT = 100000
NUM_REL = 200
D = 128
VOCAB = NUM_ENT + NUM_REL + 3
B = 4096
ARITY = 9
NUM_NEG = 128

_LN_BLOCK = 4096


def _ln_body(x_ref, s_ref, b_ref, o_ref):
    x = x_ref[...]
    mu = jnp.mean(x, axis=-1, keepdims=True)
    var = jnp.mean((x - mu) ** 2, axis=-1, keepdims=True)
    o_ref[...] = (x - mu) * lax.rsqrt(var + 1e-5) * s_ref[...] + b_ref[...]


def _layernorm_table(embs, ln_scale, ln_bias):
    n = embs.shape[0]
    grid = (n + _LN_BLOCK - 1) // _LN_BLOCK
    return pl.pallas_call(
        _ln_body,
        grid=(grid,),
        in_specs=[
            pl.BlockSpec((_LN_BLOCK, D), lambda i: (i, 0)),
            pl.BlockSpec((1, D), lambda i: (0, 0)),
            pl.BlockSpec((1, D), lambda i: (0, 0)),
        ],
        out_specs=pl.BlockSpec((_LN_BLOCK, D), lambda i: (i, 0)),
        out_shape=jax.ShapeDtypeStruct((n, D), jnp.float32),
    )(embs, ln_scale.reshape(1, D), ln_bias.reshape(1, D))


def _make_sc_score():
    info = plsc.get_sparse_core_info()
    nc, ns = info.num_cores, info.num_subcores
    nw = nc * ns
    rpw = B // nw
    mesh = plsc.VectorSubcoreMesh(core_axis_name="c", subcore_axis_name="s")

    tile = 8
    ntiles = rpw // tile

    @functools.partial(
        pl.kernel,
        mesh=mesh,
        out_type=jax.ShapeDtypeStruct((B, NUM_NEG), jnp.float32),
        scratch_types=[
            pltpu.VMEM((rpw, NUM_NEG), jnp.int32),
            pltpu.VMEM((rpw * ARITY,), jnp.int32),
            pltpu.VMEM((2, tile * ARITY, D), jnp.float32),
            pltpu.VMEM((4, NUM_NEG, D), jnp.float32),
            pltpu.VMEM((rpw, NUM_NEG), jnp.float32),
            pltpu.SemaphoreType.DMA((2,)),
            pltpu.SemaphoreType.DMA((4,)),
        ],
    )
    def sc_score(unity, tok_idx, neg_idx, out, negidx_v, tokidx_v,
                 tokrows_v, negrows_v, score_v, semt, semn):
        wid = lax.axis_index("s") * nc + lax.axis_index("c")
        base = wid * rpw
        pltpu.sync_copy(neg_idx.at[pl.ds(base, rpw)], negidx_v)
        pltpu.sync_copy(tok_idx.at[pl.ds(base * ARITY, rpw * ARITY)],
                        tokidx_v)

        def issue_neg(b, s):
            pltpu.make_async_copy(unity.at[negidx_v.at[b]], negrows_v.at[s],
                                  semn.at[s]).start()

        def issue_tok(t, ts):
            pltpu.make_async_copy(
                unity.at[tokidx_v.at[pl.ds(t * tile * ARITY, tile * ARITY)]],
                tokrows_v.at[ts], semt.at[ts]).start()

        issue_tok(0, 0)
        for i in range(3):
            issue_neg(i, i)

        def row_body(b, _):
            s = b & 3
            r = b & (tile - 1)
            t = lax.shift_right_logical(b, 3)
            ts = t & 1

            @pl.when(r == 0)
            def _tok_turnover():
                pltpu.make_async_copy(
                    unity.at[tokidx_v.at[pl.ds(t * tile * ARITY,
                                               tile * ARITY)]],
                    tokrows_v.at[ts], semt.at[ts]).wait()

                @pl.when(t + 1 < ntiles)
                def _tok_prefetch():
                    issue_tok(t + 1, 1 - ts)

            @pl.when(b + 3 < rpw)
            def _neg_prefetch():
                issue_neg(b + 3, (b + 3) & 3)

            pltpu.make_async_copy(unity.at[negidx_v.at[b]], negrows_v.at[s],
                                  semn.at[s]).wait()

            h = []
            for k in range(D // 16):
                acc = tokrows_v[ts, r * ARITY, pl.ds(k * 16, 16)]
                for tt in range(1, ARITY):
                    acc = acc + tokrows_v[ts, r * ARITY + tt,
                                          pl.ds(k * 16, 16)]
                h.append(acc * (1.0 / ARITY))

            lane = lax.broadcasted_iota(jnp.int32, (16,), 0)
            bitrev = (0, 8, 4, 12, 2, 10, 6, 14, 1, 9, 5, 13, 3, 11, 7, 15)

            def _treesum(vals):
                while len(vals) > 1:
                    vals = [vals[2 * i] + vals[2 * i + 1]
                            for i in range(len(vals) // 2)]
                return vals[0]

            def _merge(a, bb, st):
                sel = (lane & st) == 0
                m = jnp.where(sel, a, bb)
                w = jnp.where(sel, bb, a)
                return m + jnp.take_along_axis(w, lane ^ st, axis=0)

            @plsc.parallel_loop(0, NUM_NEG // 16, step=1)
            def neg_body(g):
                cur = []
                for jj in bitrev:
                    j = g * 16 + jj
                    cur.append(_treesum([
                        h[k] * negrows_v[s, j, pl.ds(k * 16, 16)]
                        for k in range(D // 16)]))
                for st in (8, 4, 2, 1):
                    cur = [_merge(cur[2 * i], cur[2 * i + 1], st)
                           for i in range(len(cur) // 2)]
                score_v[b, pl.ds(g * 16, 16)] = cur[0]

            return _

        lax.fori_loop(0, rpw, row_body, None)
        pltpu.sync_copy(score_v, out.at[pl.ds(base, rpw)])

    return sc_score


_sc_score = _make_sc_score()


@jax.jit
def kernel(batch_tokens, neg_target_index, embs, ln_scale, ln_bias):
    unity = _layernorm_table(embs, ln_scale, ln_bias)
    tok = batch_tokens.astype(jnp.int32).reshape(B * ARITY)
    neg = neg_target_index.astype(jnp.int32)
    return _sc_score(unity, tok, neg)


# speedup vs baseline: 1.1500x; 1.0199x over previous
import functools

import jax
import jax.numpy as jnp
from jax import lax
from jax.experimental import pallas as pl
from jax.experimental.pallas import tpu as pltpu
from jax.experimental.pallas import tpu_sc as plsc

NUM_ENT = 100000
NUM_REL = 200
D = 128
VOCAB = NUM_ENT + NUM_REL + 3
B = 4096
ARITY = 9
NUM_NEG = 128

_LN_BLOCK = 8192


def _ln_body(x_ref, s_ref, b_ref, o_ref):
    x = x_ref[...]
    mu = jnp.mean(x, axis=-1, keepdims=True)
    var = jnp.mean((x - mu) ** 2, axis=-1, keepdims=True)
    o_ref[...] = (x - mu) * lax.rsqrt(var + 1e-5) * s_ref[...] + b_ref[...]


def _layernorm_table(embs, ln_scale, ln_bias):
    n = embs.shape[0]
    grid = (n + _LN_BLOCK - 1) // _LN_BLOCK
    return pl.pallas_call(
        _ln_body,
        grid=(grid,),
        in_specs=[
            pl.BlockSpec((_LN_BLOCK, D), lambda i: (i, 0)),
            pl.BlockSpec((1, D), lambda i: (0, 0)),
            pl.BlockSpec((1, D), lambda i: (0, 0)),
        ],
        out_specs=pl.BlockSpec((_LN_BLOCK, D), lambda i: (i, 0)),
        out_shape=jax.ShapeDtypeStruct((n, D), jnp.float32),
    )(embs, ln_scale.reshape(1, D), ln_bias.reshape(1, D))


def _make_sc_score():
    info = plsc.get_sparse_core_info()
    nc, ns = info.num_cores, info.num_subcores
    nw = nc * ns
    rpw = B // nw
    mesh = plsc.VectorSubcoreMesh(core_axis_name="c", subcore_axis_name="s")

    tile = 8
    ntiles = rpw // tile

    @functools.partial(
        pl.kernel,
        mesh=mesh,
        out_type=jax.ShapeDtypeStruct((B, NUM_NEG), jnp.float32),
        scratch_types=[
            pltpu.VMEM((rpw, NUM_NEG), jnp.int32),
            pltpu.VMEM((rpw * ARITY,), jnp.int32),
            pltpu.VMEM((2, tile * ARITY, D), jnp.float32),
            pltpu.VMEM((4, NUM_NEG, D), jnp.float32),
            pltpu.VMEM((rpw, NUM_NEG), jnp.float32),
            pltpu.SemaphoreType.DMA((2,)),
            pltpu.SemaphoreType.DMA((4,)),
        ],
    )
    def sc_score(unity, tok_idx, neg_idx, out, negidx_v, tokidx_v,
                 tokrows_v, negrows_v, score_v, semt, semn):
        wid = lax.axis_index("s") * nc + lax.axis_index("c")
        base = wid * rpw
        pltpu.sync_copy(neg_idx.at[pl.ds(base, rpw)], negidx_v)
        pltpu.sync_copy(tok_idx.at[pl.ds(base * ARITY, rpw * ARITY)],
                        tokidx_v)

        def issue_neg(b, s):
            pltpu.make_async_copy(unity.at[negidx_v.at[b]], negrows_v.at[s],
                                  semn.at[s]).start()

        def issue_tok(t, ts):
            pltpu.make_async_copy(
                unity.at[tokidx_v.at[pl.ds(t * tile * ARITY, tile * ARITY)]],
                tokrows_v.at[ts], semt.at[ts]).start()

        issue_tok(0, 0)
        for i in range(3):
            issue_neg(i, i)

        def row_body(b, _):
            s = b & 3
            r = b & (tile - 1)
            t = lax.shift_right_logical(b, 3)
            ts = t & 1

            @pl.when(r == 0)
            def _tok_turnover():
                pltpu.make_async_copy(
                    unity.at[tokidx_v.at[pl.ds(t * tile * ARITY,
                                               tile * ARITY)]],
                    tokrows_v.at[ts], semt.at[ts]).wait()

                @pl.when(t + 1 < ntiles)
                def _tok_prefetch():
                    issue_tok(t + 1, 1 - ts)

            @pl.when(b + 3 < rpw)
            def _neg_prefetch():
                issue_neg(b + 3, (b + 3) & 3)

            pltpu.make_async_copy(unity.at[negidx_v.at[b]], negrows_v.at[s],
                                  semn.at[s]).wait()

            h = []
            for k in range(D // 16):
                acc = tokrows_v[ts, r * ARITY, pl.ds(k * 16, 16)]
                for tt in range(1, ARITY):
                    acc = acc + tokrows_v[ts, r * ARITY + tt,
                                          pl.ds(k * 16, 16)]
                h.append(acc * (1.0 / ARITY))

            lane = lax.broadcasted_iota(jnp.int32, (16,), 0)
            bitrev = (0, 8, 4, 12, 2, 10, 6, 14, 1, 9, 5, 13, 3, 11, 7, 15)

            def _treesum(vals):
                while len(vals) > 1:
                    vals = [vals[2 * i] + vals[2 * i + 1]
                            for i in range(len(vals) // 2)]
                return vals[0]

            def _merge(a, bb, st):
                sel = (lane & st) == 0
                m = jnp.where(sel, a, bb)
                w = jnp.where(sel, bb, a)
                return m + jnp.take_along_axis(w, lane ^ st, axis=0)

            @plsc.parallel_loop(0, NUM_NEG // 16, step=1)
            def neg_body(g):
                cur = []
                for jj in bitrev:
                    j = g * 16 + jj
                    cur.append(_treesum([
                        h[k] * negrows_v[s, j, pl.ds(k * 16, 16)]
                        for k in range(D // 16)]))
                for st in (8, 4, 2, 1):
                    cur = [_merge(cur[2 * i], cur[2 * i + 1], st)
                           for i in range(len(cur) // 2)]
                score_v[b, pl.ds(g * 16, 16)] = cur[0]

            return _

        lax.fori_loop(0, rpw, row_body, None)
        pltpu.sync_copy(score_v, out.at[pl.ds(base, rpw)])

    return sc_score


_sc_score = _make_sc_score()


@jax.jit
def kernel(batch_tokens, neg_target_index, embs, ln_scale, ln_bias):
    unity = _layernorm_table(embs, ln_scale, ln_bias)
    tok = batch_tokens.astype(jnp.int32).reshape(B * ARITY)
    neg = neg_target_index.astype(jnp.int32)
    return _sc_score(unity, tok, neg)


# speedup vs baseline: 1.1504x; 1.0004x over previous
import functools

import jax
import jax.numpy as jnp
from jax import lax
from jax.experimental import pallas as pl
from jax.experimental.pallas import tpu as pltpu
from jax.experimental.pallas import tpu_sc as plsc

NUM_ENT = 100000
NUM_REL = 200
D = 128
VOCAB = NUM_ENT + NUM_REL + 3
B = 4096
ARITY = 9
NUM_NEG = 128

_LN_BLOCK = 16384


def _ln_body(x_ref, s_ref, b_ref, o_ref):
    x = x_ref[...]
    mu = jnp.mean(x, axis=-1, keepdims=True)
    var = jnp.mean((x - mu) ** 2, axis=-1, keepdims=True)
    o_ref[...] = (x - mu) * lax.rsqrt(var + 1e-5) * s_ref[...] + b_ref[...]


def _layernorm_table(embs, ln_scale, ln_bias):
    n = embs.shape[0]
    grid = (n + _LN_BLOCK - 1) // _LN_BLOCK
    return pl.pallas_call(
        _ln_body,
        grid=(grid,),
        in_specs=[
            pl.BlockSpec((_LN_BLOCK, D), lambda i: (i, 0)),
            pl.BlockSpec((1, D), lambda i: (0, 0)),
            pl.BlockSpec((1, D), lambda i: (0, 0)),
        ],
        out_specs=pl.BlockSpec((_LN_BLOCK, D), lambda i: (i, 0)),
        out_shape=jax.ShapeDtypeStruct((n, D), jnp.float32),
    )(embs, ln_scale.reshape(1, D), ln_bias.reshape(1, D))


def _make_sc_score():
    info = plsc.get_sparse_core_info()
    nc, ns = info.num_cores, info.num_subcores
    nw = nc * ns
    rpw = B // nw
    mesh = plsc.VectorSubcoreMesh(core_axis_name="c", subcore_axis_name="s")

    tile = 8
    ntiles = rpw // tile

    @functools.partial(
        pl.kernel,
        mesh=mesh,
        out_type=jax.ShapeDtypeStruct((B, NUM_NEG), jnp.float32),
        scratch_types=[
            pltpu.VMEM((rpw, NUM_NEG), jnp.int32),
            pltpu.VMEM((rpw * ARITY,), jnp.int32),
            pltpu.VMEM((2, tile * ARITY, D), jnp.float32),
            pltpu.VMEM((4, NUM_NEG, D), jnp.float32),
            pltpu.VMEM((rpw, NUM_NEG), jnp.float32),
            pltpu.SemaphoreType.DMA((2,)),
            pltpu.SemaphoreType.DMA((4,)),
        ],
    )
    def sc_score(unity, tok_idx, neg_idx, out, negidx_v, tokidx_v,
                 tokrows_v, negrows_v, score_v, semt, semn):
        wid = lax.axis_index("s") * nc + lax.axis_index("c")
        base = wid * rpw
        pltpu.sync_copy(neg_idx.at[pl.ds(base, rpw)], negidx_v)
        pltpu.sync_copy(tok_idx.at[pl.ds(base * ARITY, rpw * ARITY)],
                        tokidx_v)

        def issue_neg(b, s):
            pltpu.make_async_copy(unity.at[negidx_v.at[b]], negrows_v.at[s],
                                  semn.at[s]).start()

        def issue_tok(t, ts):
            pltpu.make_async_copy(
                unity.at[tokidx_v.at[pl.ds(t * tile * ARITY, tile * ARITY)]],
                tokrows_v.at[ts], semt.at[ts]).start()

        issue_tok(0, 0)
        for i in range(3):
            issue_neg(i, i)

        def row_body(b, _):
            s = b & 3
            r = b & (tile - 1)
            t = lax.shift_right_logical(b, 3)
            ts = t & 1

            @pl.when(r == 0)
            def _tok_turnover():
                pltpu.make_async_copy(
                    unity.at[tokidx_v.at[pl.ds(t * tile * ARITY,
                                               tile * ARITY)]],
                    tokrows_v.at[ts], semt.at[ts]).wait()

                @pl.when(t + 1 < ntiles)
                def _tok_prefetch():
                    issue_tok(t + 1, 1 - ts)

            @pl.when(b + 3 < rpw)
            def _neg_prefetch():
                issue_neg(b + 3, (b + 3) & 3)

            pltpu.make_async_copy(unity.at[negidx_v.at[b]], negrows_v.at[s],
                                  semn.at[s]).wait()

            h = []
            for k in range(D // 16):
                acc = tokrows_v[ts, r * ARITY, pl.ds(k * 16, 16)]
                for tt in range(1, ARITY):
                    acc = acc + tokrows_v[ts, r * ARITY + tt,
                                          pl.ds(k * 16, 16)]
                h.append(acc * (1.0 / ARITY))

            lane = lax.broadcasted_iota(jnp.int32, (16,), 0)
            bitrev = (0, 8, 4, 12, 2, 10, 6, 14, 1, 9, 5, 13, 3, 11, 7, 15)

            def _treesum(vals):
                while len(vals) > 1:
                    vals = [vals[2 * i] + vals[2 * i + 1]
                            for i in range(len(vals) // 2)]
                return vals[0]

            def _merge(a, bb, st):
                sel = (lane & st) == 0
                m = jnp.where(sel, a, bb)
                w = jnp.where(sel, bb, a)
                return m + jnp.take_along_axis(w, lane ^ st, axis=0)

            @plsc.parallel_loop(0, NUM_NEG // 16, step=1)
            def neg_body(g):
                cur = []
                for jj in bitrev:
                    j = g * 16 + jj
                    cur.append(_treesum([
                        h[k] * negrows_v[s, j, pl.ds(k * 16, 16)]
                        for k in range(D // 16)]))
                for st in (8, 4, 2, 1):
                    cur = [_merge(cur[2 * i], cur[2 * i + 1], st)
                           for i in range(len(cur) // 2)]
                score_v[b, pl.ds(g * 16, 16)] = cur[0]

            return _

        lax.fori_loop(0, rpw, row_body, None)
        pltpu.sync_copy(score_v, out.at[pl.ds(base, rpw)])

    return sc_score


_sc_score = _make_sc_score()


@jax.jit
def kernel(batch_tokens, neg_target_index, embs, ln_scale, ln_bias):
    unity = _layernorm_table(embs, ln_scale, ln_bias)
    tok = batch_tokens.astype(jnp.int32).reshape(B * ARITY)
    neg = neg_target_index.astype(jnp.int32)
    return _sc_score(unity, tok, neg)
